# baseline (speedup 1.0000x reference)
import jax
import jax.numpy as jnp
from jax.experimental import pallas as pl
from jax.experimental.pallas import tpu as pltpu

N_ROWS = 64
N_COLS = 32768
GR = 8
NG = N_ROWS // GR


def _hardmax_pipeline(x_hbm, o_hbm, xbuf, obuf, rsem, wsem):
    def rd(g):
        return pltpu.make_async_copy(
            x_hbm.at[pl.ds(g * GR, GR), :], xbuf.at[g], rsem.at[g])

    def wr(g):
        return pltpu.make_async_copy(
            obuf.at[g], o_hbm.at[pl.ds(g * GR, GR), :], wsem.at[g])

    for g in range(NG):
        rd(g).start()

    for g in range(NG):
        rd(g).wait()
        xb = xbuf[g]
        idx = jnp.argmax(xb, axis=1, keepdims=True)
        iota = jax.lax.broadcasted_iota(jnp.int32, (GR, N_COLS), 1)
        obuf[g] = (iota == idx).astype(jnp.int32)
        wr(g).start()

    for g in range(NG):
        wr(g).wait()


def kernel(x):
    return pl.pallas_call(
        _hardmax_pipeline,
        in_specs=[pl.BlockSpec(memory_space=pl.ANY)],
        out_specs=pl.BlockSpec(memory_space=pl.ANY),
        out_shape=jax.ShapeDtypeStruct((N_ROWS, N_COLS), jnp.int32),
        scratch_shapes=[
            pltpu.VMEM((NG, GR, N_COLS), jnp.float32),
            pltpu.VMEM((NG, GR, N_COLS), jnp.int32),
            pltpu.SemaphoreType.DMA((NG,)),
            pltpu.SemaphoreType.DMA((NG,)),
        ],
    )(x)


# speedup vs baseline: 1.0624x; 1.0624x over previous
import jax
import jax.numpy as jnp
from jax.experimental import pallas as pl
from jax.experimental.pallas import tpu as pltpu

N_ROWS = 64
N_COLS = 32768
BIG = 16
SMALL = 8
N_BIG = 3
N_SMALL = 2
SIZES = (BIG,) * N_BIG + (SMALL,) * N_SMALL
STARTS = tuple(sum(SIZES[:g]) for g in range(len(SIZES)))
NG = len(SIZES)


def _hardmax_pipeline(x_hbm, o_hbm, xb_big, ob_big, xb_sm, ob_sm, rsem, wsem):
    def bufs(g):
        if g < N_BIG:
            return xb_big.at[g], ob_big.at[g]
        return xb_sm.at[g - N_BIG], ob_sm.at[g - N_BIG]

    def rd(g):
        xb, _ = bufs(g)
        return pltpu.make_async_copy(
            x_hbm.at[pl.ds(STARTS[g], SIZES[g]), :], xb, rsem.at[g])

    def wr(g):
        _, ob = bufs(g)
        return pltpu.make_async_copy(
            ob, o_hbm.at[pl.ds(STARTS[g], SIZES[g]), :], wsem.at[g])

    for g in range(NG):
        rd(g).start()

    for g in range(NG):
        xb_ref, ob_ref = bufs(g)
        rd(g).wait()
        xb = xb_ref[...]
        idx = jnp.argmax(xb, axis=1, keepdims=True)
        iota = jax.lax.broadcasted_iota(jnp.int32, xb.shape, 1)
        ob_ref[...] = (iota == idx).astype(jnp.int32)
        wr(g).start()

    for g in range(NG):
        wr(g).wait()


def kernel(x):
    return pl.pallas_call(
        _hardmax_pipeline,
        in_specs=[pl.BlockSpec(memory_space=pl.ANY)],
        out_specs=pl.BlockSpec(memory_space=pl.ANY),
        out_shape=jax.ShapeDtypeStruct((N_ROWS, N_COLS), jnp.int32),
        scratch_shapes=[
            pltpu.VMEM((N_BIG, BIG, N_COLS), jnp.float32),
            pltpu.VMEM((N_BIG, BIG, N_COLS), jnp.int32),
            pltpu.VMEM((N_SMALL, SMALL, N_COLS), jnp.float32),
            pltpu.VMEM((N_SMALL, SMALL, N_COLS), jnp.int32),
            pltpu.SemaphoreType.DMA((NG,)),
            pltpu.SemaphoreType.DMA((NG,)),
        ],
    )(x)
